# Initial kernel scaffold; baseline (speedup 1.0000x reference)
import jax
import jax.numpy as jnp
from jax.experimental import pallas as pl


def kernel(x, Wg, bg, W1, b1, W2, b2, gamma, beta):
    raise NotImplementedError("write your pallas kernel here")


# speedup vs baseline: 2.3853x; 2.3853x over previous
import functools

import jax
import jax.numpy as jnp
from jax.experimental import pallas as pl

E = 8
TOPK = 2
D = 768
H = 512
EPS = 1e-5

BT = 512


def _moe_block(x_ref, wgt_ref, bg_ref, w1_ref, b1_ref, w2_ref, b2_ref,
               gamma_ref, beta_ref, out_ref):
    e = pl.program_id(1)
    xb = x_ref[...]

    gates = jnp.dot(xb, wgt_ref[...], preferred_element_type=jnp.float32)
    gates = gates + bg_ref[...]
    eidx = jax.lax.broadcasted_iota(jnp.int32, gates.shape, 1)
    m1 = jnp.max(gates, axis=1, keepdims=True)
    a1 = jnp.min(jnp.where(gates == m1, eidx, E), axis=1, keepdims=True)
    masked = jnp.where(eidx == a1, -jnp.inf, gates)
    m2 = jnp.max(masked, axis=1, keepdims=True)
    a2 = jnp.min(jnp.where(masked == m2, eidx, E), axis=1, keepdims=True)
    z = jnp.exp(m2 - m1)
    w_top1 = 1.0 / (1.0 + z)
    w_top2 = z / (1.0 + z)
    we = jnp.where(a1 == e, w_top1, jnp.where(a2 == e, w_top2, 0.0))

    h = jnp.dot(xb, w1_ref[0], preferred_element_type=jnp.float32) + b1_ref[0]
    h = 0.5 * h * (1.0 + jax.lax.erf(h * 0.7071067811865475))
    y = jnp.dot(h, w2_ref[0], preferred_element_type=jnp.float32) + b2_ref[0]
    y = y + xb
    mu = jnp.mean(y, axis=1, keepdims=True)
    yc = y - mu
    var = jnp.mean(yc * yc, axis=1, keepdims=True)
    y = yc * jax.lax.rsqrt(var + EPS) * gamma_ref[0] + beta_ref[0]

    contrib = we * y

    @pl.when(e == 0)
    def _():
        out_ref[...] = contrib

    @pl.when(e > 0)
    def _():
        out_ref[...] += contrib


@jax.jit
def _moe(x_flat, WgT, bg2, W1T, b1, W2T, b2, gamma, beta):
    T = x_flat.shape[0]
    grid = (T // BT, E)
    return pl.pallas_call(
        _moe_block,
        grid=grid,
        in_specs=[
            pl.BlockSpec((BT, D), lambda i, e: (i, 0)),
            pl.BlockSpec((D, E), lambda i, e: (0, 0)),
            pl.BlockSpec((1, E), lambda i, e: (0, 0)),
            pl.BlockSpec((1, D, H), lambda i, e: (e, 0, 0)),
            pl.BlockSpec((1, 1, H), lambda i, e: (e, 0, 0)),
            pl.BlockSpec((1, H, D), lambda i, e: (e, 0, 0)),
            pl.BlockSpec((1, 1, D), lambda i, e: (e, 0, 0)),
            pl.BlockSpec((1, 1, D), lambda i, e: (e, 0, 0)),
            pl.BlockSpec((1, 1, D), lambda i, e: (e, 0, 0)),
        ],
        out_specs=pl.BlockSpec((BT, D), lambda i, e: (i, 0)),
        out_shape=jax.ShapeDtypeStruct((T, D), jnp.float32),
    )(x_flat, WgT, bg2, W1T, b1, W2T, b2, gamma, beta)


def kernel(x, Wg, bg, W1, b1, W2, b2, gamma, beta):
    orig_shape = x.shape
    x_flat = x.reshape(-1, D)
    out = _moe(
        x_flat,
        Wg.T,
        bg.reshape(1, E),
        jnp.swapaxes(W1, 1, 2),
        b1.reshape(E, 1, H),
        jnp.swapaxes(W2, 1, 2),
        b2.reshape(E, 1, D),
        gamma.reshape(E, 1, D),
        beta.reshape(E, 1, D),
    )
    return out.reshape(orig_shape[:-1] + (D,))


# speedup vs baseline: 2.6756x; 1.1217x over previous
import functools

import jax
import jax.numpy as jnp
from jax.experimental import pallas as pl

E = 8
TOPK = 2
D = 768
H = 512
EPS = 1e-5

BT = 512


def _moe_block(x_ref, wgt_ref, bg_ref, w1_ref, b1_ref, w2_ref, b2_ref,
               gamma_ref, beta_ref, out_ref):
    e = pl.program_id(1)
    xb = x_ref[...]

    gates = jnp.dot(xb, wgt_ref[...], preferred_element_type=jnp.float32)
    gates = gates + bg_ref[...]
    eidx = jax.lax.broadcasted_iota(jnp.int32, gates.shape, 1)
    m1 = jnp.max(gates, axis=1, keepdims=True)
    a1 = jnp.min(jnp.where(gates == m1, eidx, E), axis=1, keepdims=True)
    masked = jnp.where(eidx == a1, -jnp.inf, gates)
    m2 = jnp.max(masked, axis=1, keepdims=True)
    a2 = jnp.min(jnp.where(masked == m2, eidx, E), axis=1, keepdims=True)
    z = jnp.exp(m2 - m1)
    w_top1 = 1.0 / (1.0 + z)
    w_top2 = z / (1.0 + z)
    we = jnp.where(a1 == e, w_top1, jnp.where(a2 == e, w_top2, 0.0))

    xb_bf = xb.astype(jnp.bfloat16)
    h = jnp.dot(xb_bf, w1_ref[0], preferred_element_type=jnp.float32) + b1_ref[0]
    h = 0.5 * h * (1.0 + jax.lax.erf(h * 0.7071067811865475))
    y = jnp.dot(h.astype(jnp.bfloat16), w2_ref[0],
                preferred_element_type=jnp.float32) + b2_ref[0]
    y = y + xb
    mu = jnp.mean(y, axis=1, keepdims=True)
    yc = y - mu
    var = jnp.mean(yc * yc, axis=1, keepdims=True)
    y = yc * jax.lax.rsqrt(var + EPS) * gamma_ref[0] + beta_ref[0]

    contrib = we * y

    @pl.when(e == 0)
    def _():
        out_ref[...] = contrib

    @pl.when(e > 0)
    def _():
        out_ref[...] += contrib


@jax.jit
def _moe(x_flat, WgT, bg2, W1T, b1, W2T, b2, gamma, beta):
    T = x_flat.shape[0]
    grid = (T // BT, E)
    return pl.pallas_call(
        _moe_block,
        grid=grid,
        in_specs=[
            pl.BlockSpec((BT, D), lambda i, e: (i, 0)),
            pl.BlockSpec((D, E), lambda i, e: (0, 0)),
            pl.BlockSpec((1, E), lambda i, e: (0, 0)),
            pl.BlockSpec((1, D, H), lambda i, e: (e, 0, 0)),
            pl.BlockSpec((1, 1, H), lambda i, e: (e, 0, 0)),
            pl.BlockSpec((1, H, D), lambda i, e: (e, 0, 0)),
            pl.BlockSpec((1, 1, D), lambda i, e: (e, 0, 0)),
            pl.BlockSpec((1, 1, D), lambda i, e: (e, 0, 0)),
            pl.BlockSpec((1, 1, D), lambda i, e: (e, 0, 0)),
        ],
        out_specs=pl.BlockSpec((BT, D), lambda i, e: (i, 0)),
        out_shape=jax.ShapeDtypeStruct((T, D), jnp.float32),
    )(x_flat, WgT, bg2, W1T, b1, W2T, b2, gamma, beta)


def kernel(x, Wg, bg, W1, b1, W2, b2, gamma, beta):
    orig_shape = x.shape
    x_flat = x.reshape(-1, D)
    out = _moe(
        x_flat,
        Wg.T,
        bg.reshape(1, E),
        jnp.swapaxes(W1, 1, 2).astype(jnp.bfloat16),
        b1.reshape(E, 1, H),
        jnp.swapaxes(W2, 1, 2).astype(jnp.bfloat16),
        b2.reshape(E, 1, D),
        gamma.reshape(E, 1, D),
        beta.reshape(E, 1, D),
    )
    return out.reshape(orig_shape[:-1] + (D,))


# speedup vs baseline: 2.7737x; 1.0366x over previous
import functools

import jax
import jax.numpy as jnp
from jax.experimental import pallas as pl
from jax.experimental.pallas import tpu as pltpu

E = 8
TOPK = 2
D = 768
H = 512
EPS = 1e-5

BT = 512


def _moe_block(x_ref, wgt_ref, bg_ref, w1_ref, b1_ref, w2_ref, b2_ref,
               gamma_ref, beta_ref, out_ref, we_ref):
    e = pl.program_id(1)
    xb = x_ref[...]

    @pl.when(e == 0)
    def _():
        gates = jnp.dot(xb, wgt_ref[...], preferred_element_type=jnp.float32)
        gates = gates + bg_ref[...]
        eidx = jax.lax.broadcasted_iota(jnp.int32, gates.shape, 1)
        m1 = jnp.max(gates, axis=1, keepdims=True)
        a1 = jnp.min(jnp.where(gates == m1, eidx, E), axis=1, keepdims=True)
        masked = jnp.where(eidx == a1, -jnp.inf, gates)
        m2 = jnp.max(masked, axis=1, keepdims=True)
        a2 = jnp.min(jnp.where(masked == m2, eidx, E), axis=1, keepdims=True)
        z = jnp.exp(m2 - m1)
        w_top1 = 1.0 / (1.0 + z)
        w_top2 = z / (1.0 + z)
        we_ref[...] = jnp.where(
            eidx == a1, w_top1, jnp.where(eidx == a2, w_top2, 0.0))

    we_all = we_ref[...]
    eidx2 = jax.lax.broadcasted_iota(jnp.int32, we_all.shape, 1)
    we = jnp.sum(jnp.where(eidx2 == e, we_all, 0.0), axis=1, keepdims=True)

    xb_bf = xb.astype(jnp.bfloat16)
    h = jnp.dot(xb_bf, w1_ref[0], preferred_element_type=jnp.float32) + b1_ref[0]
    h = 0.5 * h * (1.0 + jax.lax.erf(h * 0.7071067811865475))
    y = jnp.dot(h.astype(jnp.bfloat16), w2_ref[0],
                preferred_element_type=jnp.float32) + b2_ref[0]
    y = y + xb
    mu = jnp.mean(y, axis=1, keepdims=True)
    yc = y - mu
    var = jnp.mean(yc * yc, axis=1, keepdims=True)
    y = yc * jax.lax.rsqrt(var + EPS) * gamma_ref[0] + beta_ref[0]

    contrib = we * y

    @pl.when(e == 0)
    def _():
        out_ref[...] = contrib

    @pl.when(e > 0)
    def _():
        out_ref[...] += contrib


@jax.jit
def _moe(x_flat, WgT, bg2, W1T, b1, W2T, b2, gamma, beta):
    T = x_flat.shape[0]
    grid = (T // BT, E)
    return pl.pallas_call(
        _moe_block,
        grid=grid,
        in_specs=[
            pl.BlockSpec((BT, D), lambda i, e: (i, 0)),
            pl.BlockSpec((D, E), lambda i, e: (0, 0)),
            pl.BlockSpec((1, E), lambda i, e: (0, 0)),
            pl.BlockSpec((1, D, H), lambda i, e: (e, 0, 0)),
            pl.BlockSpec((1, 1, H), lambda i, e: (e, 0, 0)),
            pl.BlockSpec((1, H, D), lambda i, e: (e, 0, 0)),
            pl.BlockSpec((1, 1, D), lambda i, e: (e, 0, 0)),
            pl.BlockSpec((1, 1, D), lambda i, e: (e, 0, 0)),
            pl.BlockSpec((1, 1, D), lambda i, e: (e, 0, 0)),
        ],
        out_specs=pl.BlockSpec((BT, D), lambda i, e: (i, 0)),
        out_shape=jax.ShapeDtypeStruct((T, D), jnp.float32),
        scratch_shapes=[pltpu.VMEM((BT, E), jnp.float32)],
    )(x_flat, WgT, bg2, W1T, b1, W2T, b2, gamma, beta)


def kernel(x, Wg, bg, W1, b1, W2, b2, gamma, beta):
    orig_shape = x.shape
    x_flat = x.reshape(-1, D)
    out = _moe(
        x_flat,
        Wg.T,
        bg.reshape(1, E),
        jnp.swapaxes(W1, 1, 2).astype(jnp.bfloat16),
        b1.reshape(E, 1, H),
        jnp.swapaxes(W2, 1, 2).astype(jnp.bfloat16),
        b2.reshape(E, 1, D),
        gamma.reshape(E, 1, D),
        beta.reshape(E, 1, D),
    )
    return out.reshape(orig_shape[:-1] + (D,))
